# baseline (speedup 1.0000x reference)
import functools

import jax
import jax.numpy as jnp
from jax import lax
from jax.experimental import pallas as pl
from jax.experimental.pallas import tpu as pltpu
from jax.experimental.pallas import tpu_sc as plsc

_B = 2
_Q = 100
_NPTS = 50000
_M = 30
_NCLS = 20
_NPOINT = 12288
_EOS_COEF = 0.1
_W_CE = 2.0
_W_MASK = 5.0
_W_DICE = 5.0

_NW = 32
_JPT = (_B * _NPOINT) // _NW
_CH = 128
_NCH = _JPT // _CH


def _sc_gather(pred_units, tgt_flat, idx_flat):
    mesh = plsc.VectorSubcoreMesh(core_axis_name="c", subcore_axis_name="s")

    @functools.partial(
        pl.kernel,
        mesh=mesh,
        compiler_params=pltpu.CompilerParams(
            needs_layout_passes=False, use_tc_tiling_on_sc=False),
        out_type=(
            jax.ShapeDtypeStruct((_B, _M, _NPOINT), jnp.float32),
            jax.ShapeDtypeStruct((_B, _M, _NPOINT), jnp.float32),
        ),
        scratch_types=[
            pltpu.VMEM((_NCH, _CH), jnp.int32),
            pltpu.VMEM((_M, _CH), jnp.int32),
            pltpu.VMEM((3, _CH), jnp.int32),
            pltpu.VMEM((3 * _CH, 16), jnp.float32),
            pltpu.VMEM((_M, _JPT), jnp.float32),
            pltpu.VMEM((_M, _JPT), jnp.float32),
            pltpu.SemaphoreType.DMA,
            pltpu.SemaphoreType.DMA,
        ],
    )
    def k(pred_hbm, tgt_hbm, idx_hbm, gx_hbm, gt_hbm,
          pidx_v, tidx_v, uidx_v, units_v, gx_v, gt_v, sem_p, sem_t):
        wid = lax.axis_index("s") * 2 + lax.axis_index("c")
        b = wid // 16
        jbase = (wid % 16) * _JPT

        pltpu.sync_copy(idx_hbm.at[pl.ds(wid * _NCH, _NCH), :], pidx_v)
        rowoff = b * _NPTS
        for kk in range(_NCH):
            for i in range(_CH // 16):
                sl = pl.ds(i * 16, 16)
                pidx_v[kk, sl] = pidx_v[kk, sl] + rowoff

        lane = lax.iota(jnp.int32, 16)

        def chunk_body(kc, _):
            koff = pl.multiple_of(kc * _CH, _CH)
            for i in range(_CH // 16):
                sl = pl.ds(i * 16, 16)
                r = pidx_v[kc, sl]
                u0 = r * 6 + lax.shift_right_logical(r, 2)
                uidx_v[0, sl] = u0
                uidx_v[1, sl] = u0 + 1
                uidx_v[2, sl] = u0 + 2
            cps = [
                pltpu.async_copy(
                    pred_hbm.at[uidx_v.at[u]],
                    units_v.at[pl.ds(u * _CH, _CH), :], sem_p)
                for u in range(3)
            ]
            for m in range(_M):
                moff = (b * (_M - 1) + m) * _NPTS
                for i in range(_CH // 16):
                    sl = pl.ds(i * 16, 16)
                    tidx_v[m, sl] = pidx_v[kc, sl] + moff
            tcps = [
                pltpu.async_copy(
                    tgt_hbm.at[tidx_v.at[m]],
                    gt_v.at[m, pl.ds(koff, _CH)], sem_t)
                for m in range(_M)
            ]
            for c in cps:
                c.wait()
            for g in range(_CH // 16):
                sl = pl.ds(g * 16, 16)
                jv = lane + (g * 16)
                off = (pidx_v[kc, sl] & 3) * 4
                for m in range(_M):
                    w = off + m
                    hi = w & (-16)
                    rows = hi * 8 + jv
                    cols = w - hi
                    gx_v[m, pl.ds(koff + g * 16, 16)] = plsc.load_gather(
                        units_v, [rows, cols])
            for c in tcps:
                c.wait()
            return ()

        lax.fori_loop(0, _NCH, chunk_body, (), unroll=False)

        outs = []
        for m in range(_M):
            outs.append(pltpu.async_copy(
                gx_v.at[m], gx_hbm.at[b, m, pl.ds(jbase, _JPT)], sem_p))
            outs.append(pltpu.async_copy(
                gt_v.at[m], gt_hbm.at[b, m, pl.ds(jbase, _JPT)], sem_t))
        for c in outs:
            c.wait()

    return k(pred_units, tgt_flat, idx_flat)


def _loss_body(gx_ref, gt_ref, plog_ref, tgt_ref, cw_ref, out_ref):
    dice_sum = jnp.float32(0.0)
    bce_sum = jnp.float32(0.0)
    for b in range(_B):
        x = gx_ref[b]
        t = gt_ref[b]
        s = 1.0 / (1.0 + jnp.exp(-x))
        sum_s = jnp.sum(s, axis=1, keepdims=True)
        sum_t = jnp.sum(t, axis=1, keepdims=True)
        sum_st = jnp.sum(s * t, axis=1, keepdims=True)
        dice = 1.0 - (2.0 * sum_st + 1.0) / (sum_s + sum_t + 1.0)
        dice_sum = dice_sum + jnp.sum(dice)
        bce = (jnp.maximum(x, 0.0) - x * t
               + jnp.log1p(jnp.exp(-jnp.abs(x))))
        bce_sum = bce_sum + jnp.sum(bce)
    num_masks = float(_B * _M)
    loss_dice = dice_sum / num_masks
    loss_mask = bce_sum / (float(_NPOINT) * num_masks)

    xl = plog_ref[...]
    mx = jnp.max(xl, axis=1, keepdims=True)
    lse = jnp.log(jnp.sum(jnp.exp(xl - mx), axis=1, keepdims=True)) + mx
    lanes = lax.broadcasted_iota(jnp.int32, xl.shape, 1)
    oh = (lanes == tgt_ref[...]).astype(jnp.float32)
    sel = jnp.sum(oh * xl, axis=1, keepdims=True)
    nll = lse - sel
    w = jnp.sum(oh * cw_ref[...], axis=1, keepdims=True)
    loss_ce = jnp.sum(nll * w) / jnp.sum(w)

    l3 = lax.broadcasted_iota(jnp.int32, (1, 128), 1)
    out_ref[...] = (jnp.where(l3 == 0, _W_CE * loss_ce, 0.0)
                    + jnp.where(l3 == 1, _W_MASK * loss_mask, 0.0)
                    + jnp.where(l3 == 2, _W_DICE * loss_dice, 0.0))


def _tc_loss(gx, gt, plog, tgtf, cw, interpret=False):
    return pl.pallas_call(
        _loss_body,
        out_shape=jax.ShapeDtypeStruct((1, 128), jnp.float32),
        interpret=interpret,
    )(gx, gt, plog, tgtf, cw)


def kernel(pred_logits, pred_masks, target_masks, target_classes,
           sampled_idx, class_weights):
    pred_units = pred_masks.astype(jnp.float32).reshape(_B * _NPTS * _Q // 16, 16)
    tgt_flat = target_masks.astype(jnp.float32).reshape(-1)
    idx_flat = sampled_idx.astype(jnp.int32).reshape(-1, _CH)
    gx, gt = _sc_gather(pred_units, tgt_flat, idx_flat)

    tgt_full = jnp.full((_B, _Q), _NCLS, jnp.int32)
    tgt_full = tgt_full.at[:, :_M].set(target_classes.astype(jnp.int32))
    tgtf = tgt_full.reshape(_B * _Q, 1)
    plog = jnp.pad(
        pred_logits.astype(jnp.float32).reshape(_B * _Q, _NCLS + 1),
        ((0, 0), (0, 128 - (_NCLS + 1))), constant_values=-1e30)
    cw = jnp.pad(class_weights.astype(jnp.float32),
                 (0, 128 - (_NCLS + 1))).reshape(1, 128)
    out = _tc_loss(gx, gt, plog, tgtf, cw)
    return out[0, :3]


# speedup vs baseline: 1.0126x; 1.0126x over previous
import functools

import jax
import jax.numpy as jnp
from jax import lax
from jax.experimental import pallas as pl
from jax.experimental.pallas import tpu as pltpu
from jax.experimental.pallas import tpu_sc as plsc

_B = 2
_Q = 100
_NPTS = 50000
_M = 30
_NCLS = 20
_NPOINT = 12288
_EOS_COEF = 0.1
_W_CE = 2.0
_W_MASK = 5.0
_W_DICE = 5.0

_NW = 32
_JPT = (_B * _NPOINT) // _NW
_CH = 128
_NCH = _JPT // _CH


def _sc_gather(pred_units, tgt_flat, idx_flat):
    nunits = pred_units.shape[0]
    mesh = plsc.VectorSubcoreMesh(core_axis_name="c", subcore_axis_name="s")

    @functools.partial(
        pl.kernel,
        mesh=mesh,
        compiler_params=pltpu.CompilerParams(needs_layout_passes=False),
        out_type=(
            jax.ShapeDtypeStruct((_B, _M, _NPOINT), jnp.float32),
            jax.ShapeDtypeStruct((_B, _M, _NPOINT), jnp.float32),
        ),
        scratch_types=[
            pltpu.VMEM((_NCH, _CH), jnp.int32),
            pltpu.VMEM((_M, _CH), jnp.int32),
            pltpu.VMEM((2, _CH), jnp.int32),
            pltpu.VMEM((2 * _CH, 128), jnp.float32),
            pltpu.VMEM((_M, _JPT), jnp.float32),
            pltpu.VMEM((_M, _JPT), jnp.float32),
            pltpu.SemaphoreType.DMA,
            pltpu.SemaphoreType.DMA,
        ],
    )
    def k(pred_hbm, tgt_hbm, idx_hbm, gx_hbm, gt_hbm,
          pidx_v, tidx_v, uidx_v, units_v, gx_v, gt_v, sem_p, sem_t):
        wid = lax.axis_index("s") * 2 + lax.axis_index("c")
        b = wid // 16
        jbase = (wid % 16) * _JPT

        for kk in range(_NCH):
            pltpu.sync_copy(
                idx_hbm.at[pl.ds(wid * _JPT + kk * _CH, _CH)], pidx_v.at[kk])
        rowoff = b * _NPTS
        for kk in range(_NCH):
            for i in range(_CH // 16):
                sl = pl.ds(i * 16, 16)
                pidx_v[kk, sl] = pidx_v[kk, sl] + rowoff

        lane = lax.iota(jnp.int32, 16)

        def chunk_body(kc, _):
            koff = pl.multiple_of(kc * _CH, _CH)
            for i in range(_CH // 16):
                sl = pl.ds(i * 16, 16)
                u0 = lax.shift_right_logical(pidx_v[kc, sl] * 100, 7)
                uidx_v[0, sl] = u0
                uidx_v[1, sl] = jnp.minimum(u0 + 1, nunits - 1)
            cps = [
                pltpu.async_copy(
                    pred_hbm.at[uidx_v.at[u]],
                    units_v.at[pl.ds(u * _CH, _CH), :], sem_p)
                for u in range(2)
            ]
            for m in range(_M):
                moff = (b * (_M - 1) + m) * _NPTS
                for i in range(_CH // 16):
                    sl = pl.ds(i * 16, 16)
                    tidx_v[m, sl] = pidx_v[kc, sl] + moff
            tcps = [
                pltpu.async_copy(
                    tgt_hbm.at[tidx_v.at[m]],
                    gt_v.at[m, pl.ds(koff, _CH)], sem_t)
                for m in range(_M)
            ]
            for c in cps:
                c.wait()
            for g in range(_CH // 16):
                sl = pl.ds(g * 16, 16)
                jv = lane + (g * 16)
                off = (pidx_v[kc, sl] * 100) & 127
                for m in range(_M):
                    w = off + m
                    rows = jv + (w & 128)
                    cols = w & 127
                    gx_v[m, pl.ds(koff + g * 16, 16)] = plsc.load_gather(
                        units_v, [rows, cols])
            for c in tcps:
                c.wait()
            return ()

        lax.fori_loop(0, _NCH, chunk_body, (), unroll=False)

        outs = []
        for m in range(_M):
            outs.append(pltpu.async_copy(
                gx_v.at[m], gx_hbm.at[b, m, pl.ds(jbase, _JPT)], sem_p))
            outs.append(pltpu.async_copy(
                gt_v.at[m], gt_hbm.at[b, m, pl.ds(jbase, _JPT)], sem_t))
        for c in outs:
            c.wait()

    return k(pred_units, tgt_flat, idx_flat)


def _loss_body(gx_ref, gt_ref, plog_ref, tgt_ref, cw_ref, out_ref):
    dice_sum = jnp.float32(0.0)
    bce_sum = jnp.float32(0.0)
    for b in range(_B):
        x = gx_ref[b]
        t = gt_ref[b]
        s = 1.0 / (1.0 + jnp.exp(-x))
        sum_s = jnp.sum(s, axis=1, keepdims=True)
        sum_t = jnp.sum(t, axis=1, keepdims=True)
        sum_st = jnp.sum(s * t, axis=1, keepdims=True)
        dice = 1.0 - (2.0 * sum_st + 1.0) / (sum_s + sum_t + 1.0)
        dice_sum = dice_sum + jnp.sum(dice)
        bce = (jnp.maximum(x, 0.0) - x * t
               + jnp.log1p(jnp.exp(-jnp.abs(x))))
        bce_sum = bce_sum + jnp.sum(bce)
    num_masks = float(_B * _M)
    loss_dice = dice_sum / num_masks
    loss_mask = bce_sum / (float(_NPOINT) * num_masks)

    xl = plog_ref[...]
    mx = jnp.max(xl, axis=1, keepdims=True)
    lse = jnp.log(jnp.sum(jnp.exp(xl - mx), axis=1, keepdims=True)) + mx
    lanes = lax.broadcasted_iota(jnp.int32, xl.shape, 1)
    oh = (lanes == tgt_ref[...]).astype(jnp.float32)
    sel = jnp.sum(oh * xl, axis=1, keepdims=True)
    nll = lse - sel
    w = jnp.sum(oh * cw_ref[...], axis=1, keepdims=True)
    loss_ce = jnp.sum(nll * w) / jnp.sum(w)

    l3 = lax.broadcasted_iota(jnp.int32, (1, 128), 1)
    out_ref[...] = (jnp.where(l3 == 0, _W_CE * loss_ce, 0.0)
                    + jnp.where(l3 == 1, _W_MASK * loss_mask, 0.0)
                    + jnp.where(l3 == 2, _W_DICE * loss_dice, 0.0))


def _tc_loss(gx, gt, plog, tgtf, cw, interpret=False):
    return pl.pallas_call(
        _loss_body,
        out_shape=jax.ShapeDtypeStruct((1, 128), jnp.float32),
        interpret=interpret,
    )(gx, gt, plog, tgtf, cw)


def kernel(pred_logits, pred_masks, target_masks, target_classes,
           sampled_idx, class_weights):
    pred_units = pred_masks.astype(jnp.float32).reshape(_B * _NPTS * _Q // 128, 128)
    tgt_flat = target_masks.astype(jnp.float32).reshape(-1)
    idx_flat = sampled_idx.astype(jnp.int32).reshape(-1)
    gx, gt = _sc_gather(pred_units, tgt_flat, idx_flat)

    tgt_full = jnp.full((_B, _Q), _NCLS, jnp.int32)
    tgt_full = tgt_full.at[:, :_M].set(target_classes.astype(jnp.int32))
    tgtf = tgt_full.reshape(_B * _Q, 1)
    plog = jnp.pad(
        pred_logits.astype(jnp.float32).reshape(_B * _Q, _NCLS + 1),
        ((0, 0), (0, 128 - (_NCLS + 1))), constant_values=-1e30)
    cw = jnp.pad(class_weights.astype(jnp.float32),
                 (0, 128 - (_NCLS + 1))).reshape(1, 128)
    out = _tc_loss(gx, gt, plog, tgtf, cw)
    return out[0, :3]
